# Initial kernel scaffold; baseline (speedup 1.0000x reference)
import functools

import jax
import jax.numpy as jnp
from jax import lax
from jax.experimental import pallas as pl
from jax.experimental.pallas import tpu as pltpu
from jax.experimental.pallas import tpu_sc as plsc

N = 10000
E = 320000
K = 10
H = 128

NC = 2
NS = 16
NW = NC * NS
L = 16

EPW = E // NW
B = 80
NBATCH = EPW // B
RPS = N // NS
ZR = 125


def _sc_body(h_hbm, src_hbm, dst_hbm, epre_hbm, part_hbm, spart_hbm,
             srcv, dstv, eprev, rows, zbuf, zbuf2, acc, sacc, sem):
    c = lax.axis_index("c")
    s = lax.axis_index("s")
    wid = c * NS + s

    def _zero_body(i, _):
        for j in range(H // L):
            zbuf[i, pl.ds(j * L, L)] = jnp.zeros((L,), jnp.float32)
        zbuf2[i, pl.ds(0, L)] = jnp.zeros((L,), jnp.float32)
        return 0
    lax.fori_loop(0, ZR, _zero_body, 0)
    for t in range(RPS // ZR):
        pltpu.sync_copy(zbuf, acc.at[pl.ds(s * RPS + t * ZR, ZR)])
        pltpu.sync_copy(zbuf2, sacc.at[pl.ds(s * RPS + t * ZR, ZR)])
    plsc.subcore_barrier()

    def _edge_body(it, _):
        base = wid * EPW + it * B
        pltpu.sync_copy(src_hbm.at[pl.ds(base, B)], srcv)
        pltpu.sync_copy(dst_hbm.at[pl.ds(base, B)], dstv)
        pltpu.sync_copy(epre_hbm.at[pl.ds(base, B)], eprev)
        pltpu.async_copy(h_hbm.at[srcv], rows, sem).wait()

        def _scale_body(i, _):
            ridx = jnp.full((L,), i, dtype=jnp.int32)
            cidx = jnp.zeros((L,), dtype=jnp.int32)
            wv = plsc.load_gather(eprev, [ridx, cidx])
            for j in range(H // L):
                rows[i, pl.ds(j * L, L)] = rows[i, pl.ds(j * L, L)] * wv
            return 0
        lax.fori_loop(0, B, _scale_body, 0)

        pltpu.sync_copy(rows, acc.at[dstv], add=True)
        pltpu.sync_copy(eprev, sacc.at[dstv], add=True)
        return 0
    lax.fori_loop(0, NBATCH, _edge_body, 0)

    plsc.subcore_barrier()

    for t in range(RPS // ZR):
        r0 = s * RPS + t * ZR
        pltpu.sync_copy(acc.at[pl.ds(r0, ZR)], part_hbm.at[c, pl.ds(r0, ZR)])
        pltpu.sync_copy(sacc.at[pl.ds(r0, ZR)], spart_hbm.at[c, pl.ds(r0, ZR)])


_sc_call = pl.kernel(
    _sc_body,
    out_type=[
        jax.ShapeDtypeStruct((NC, N, H), jnp.float32),
        jax.ShapeDtypeStruct((NC, N, L), jnp.float32),
    ],
    mesh=plsc.VectorSubcoreMesh(core_axis_name="c", subcore_axis_name="s"),
    scratch_types=[
        pltpu.VMEM((B,), jnp.int32),
        pltpu.VMEM((B,), jnp.int32),
        pltpu.VMEM((B, L), jnp.float32),
        pltpu.VMEM((B, H), jnp.float32),
        pltpu.VMEM((ZR, H), jnp.float32),
        pltpu.VMEM((ZR, L), jnp.float32),
        pltpu.VMEM_SHARED((N, H), jnp.float32),
        pltpu.VMEM_SHARED((N, L), jnp.float32),
        pltpu.SemaphoreType.DMA,
    ],
)


_EBLK = 32000

def _pre_body(d_ref, et_ref, out_ref):
    et0 = et_ref[:, 0:1]
    nd = d_ref[...] * et0
    w = nd * et0
    out_ref[...] = jnp.concatenate(
        [w, nd, jnp.zeros((_EBLK, L - 2), jnp.float32)], axis=1)


def _preprocess(d, e_type):
    return pl.pallas_call(
        _pre_body,
        grid=(E // _EBLK,),
        in_specs=[
            pl.BlockSpec((_EBLK, 1), lambda i: (i, 0)),
            pl.BlockSpec((_EBLK, 2), lambda i: (i, 0)),
        ],
        out_specs=pl.BlockSpec((_EBLK, L), lambda i: (i, 0)),
        out_shape=jax.ShapeDtypeStruct((E, L), jnp.float32),
    )(d, e_type)


_RBLK = 1000

def _dense_body(part_ref, spart_ref, label_ref, wl1_ref, bl1_ref, wl2_ref,
                bl2_ref, wt3_ref, bt3_ref, wt4_ref, out_ref):
    f32 = jnp.float32
    n1h = part_ref[0] + part_ref[1]
    s2 = spart_ref[0, :, 1:2] + spart_ref[1, :, 1:2]
    w4r = jnp.maximum(wt4_ref[...], 0.0)
    v = lax.dot_general(wt3_ref[...], w4r,
                        (((1,), (0,)), ((), ())),
                        preferred_element_type=f32)
    t3 = lax.dot_general(s2, v, (((1,), (1,)), ((), ())),
                         preferred_element_type=f32)
    l1 = lax.dot_general(label_ref[...], wl1_ref[...],
                         (((1,), (1,)), ((), ())),
                         preferred_element_type=f32)
    l2 = lax.dot_general(n1h, wl2_ref[...],
                         (((1,), (1,)), ((), ())),
                         preferred_element_type=f32)
    bias = (bl1_ref[...] + bl2_ref[...] + bt3_ref[...])[None, :]
    out_ref[...] = jnp.maximum(l1 + l2 + t3 + bias, 0.0)


def _dense(part, spart, label, W_l1, b_l1, W_l2, b_l2, W_t3, b_t3, W_t4):
    full2 = lambda i: (0, 0)
    return pl.pallas_call(
        _dense_body,
        grid=(N // _RBLK,),
        in_specs=[
            pl.BlockSpec((NC, _RBLK, H), lambda i: (0, i, 0)),
            pl.BlockSpec((NC, _RBLK, L), lambda i: (0, i, 0)),
            pl.BlockSpec((_RBLK, K), lambda i: (i, 0)),
            pl.BlockSpec((H, K), full2),
            pl.BlockSpec((H,), lambda i: (0,)),
            pl.BlockSpec((H, H), full2),
            pl.BlockSpec((H,), lambda i: (0,)),
            pl.BlockSpec((H, H), full2),
            pl.BlockSpec((H,), lambda i: (0,)),
            pl.BlockSpec((H, 1), full2),
        ],
        out_specs=pl.BlockSpec((_RBLK, H), lambda i: (i, 0)),
        out_shape=jax.ShapeDtypeStruct((N, H), jnp.float32),
    )(part, spart, label, W_l1, b_l1, W_l2, b_l2, W_t3, b_t3, W_t4)


def kernel(h, label, d, e_type, src, dst, W_l1, b_l1, W_l2, b_l2,
           W_t3, b_t3, W_t4, b_t4):
    del b_t4
    epre = _preprocess(d, e_type)
    part, spart = _sc_call(h, src, dst, epre)
    return _dense(part, spart, label, W_l1, b_l1, W_l2, b_l2, W_t3, b_t3, W_t4)


# speedup vs baseline: 2.5765x; 2.5765x over previous
import functools

import jax
import jax.numpy as jnp
from jax import lax
from jax.experimental import pallas as pl
from jax.experimental.pallas import tpu as pltpu
from jax.experimental.pallas import tpu_sc as plsc

N = 10000
E = 320000
K = 10
H = 128

NC = 2
NS = 16
NW = NC * NS
L = 16

EPW = E // NW
B = 80
NBATCH = EPW // B
N_PAD = 10240
SROWS = N_PAD // NS


def _sc_body(h_hbm, src_hbm, dst_hbm, wlin_hbm, ndlin_hbm, part_hbm, spart_hbm,
             srcv, dstv, wv_b, nd_b, rows, s2loc, acc, sem):
    c = lax.axis_index("c")
    s = lax.axis_index("s")
    wid = c * NS + s

    def _zero_body(i, _):
        for j in range(H // L):
            rows[i, pl.ds(j * L, L)] = jnp.zeros((L,), jnp.float32)
        return 0
    lax.fori_loop(0, B, _zero_body, 0)

    def _zero_s2(i, _):
        s2loc[pl.ds(i * L, L)] = jnp.zeros((L,), jnp.float32)
        return 0
    lax.fori_loop(0, N_PAD // L, _zero_s2, 0)

    for t in range(SROWS // B):
        pltpu.sync_copy(rows, acc.at[pl.ds(s * SROWS + t * B, B)])
    plsc.subcore_barrier()

    def _edge_body(it, _):
        base = wid * EPW + it * B
        pltpu.sync_copy(src_hbm.at[pl.ds(base, B)], srcv)
        pltpu.sync_copy(dst_hbm.at[pl.ds(base, B)], dstv)
        pltpu.sync_copy(wlin_hbm.at[pl.ds(base, B)], wv_b)
        pltpu.sync_copy(ndlin_hbm.at[pl.ds(base, B)], nd_b)
        pltpu.async_copy(h_hbm.at[srcv], rows, sem).wait()

        def _scale_body(i, _):
            ridx = jnp.full((L,), i, dtype=jnp.int32)
            wv = plsc.load_gather(wv_b, [ridx])
            for j in range(H // L):
                rows[i, pl.ds(j * L, L)] = rows[i, pl.ds(j * L, L)] * wv
            return 0
        lax.fori_loop(0, B, _scale_body, 0)

        def _s2_body(k, _):
            idx16 = dstv[pl.ds(k * L, L)]
            v16 = nd_b[pl.ds(k * L, L)]
            plsc.addupdate_scatter(s2loc, [idx16], v16)
            return 0
        lax.fori_loop(0, B // L, _s2_body, 0)

        pltpu.sync_copy(rows, acc.at[dstv], add=True)
        return 0
    lax.fori_loop(0, NBATCH, _edge_body, 0)

    plsc.subcore_barrier()

    r0 = s * SROWS
    pltpu.sync_copy(acc.at[pl.ds(r0, SROWS)], part_hbm.at[c, pl.ds(r0, SROWS)])
    pltpu.sync_copy(s2loc, spart_hbm.at[wid])


_sc_call = pl.kernel(
    _sc_body,
    out_type=[
        jax.ShapeDtypeStruct((NC, N_PAD, H), jnp.float32),
        jax.ShapeDtypeStruct((NW, N_PAD), jnp.float32),
    ],
    mesh=plsc.VectorSubcoreMesh(core_axis_name="c", subcore_axis_name="s"),
    compiler_params=pltpu.CompilerParams(needs_layout_passes=False),
    scratch_types=[
        pltpu.VMEM((B,), jnp.int32),
        pltpu.VMEM((B,), jnp.int32),
        pltpu.VMEM((B,), jnp.float32),
        pltpu.VMEM((B,), jnp.float32),
        pltpu.VMEM((B, H), jnp.float32),
        pltpu.VMEM((N_PAD,), jnp.float32),
        pltpu.VMEM_SHARED((N_PAD, H), jnp.float32),
        pltpu.SemaphoreType.DMA,
    ],
)


_EBLK = 2000

def _pre_body(d_ref, et_ref, wlin_ref, ndlin_ref):
    et0 = et_ref[:, 0:1]
    nd = d_ref[...] * et0
    ndlin_ref[...] = nd
    wlin_ref[...] = nd * et0


def _preprocess(d, e_type):
    wlin, ndlin = pl.pallas_call(
        _pre_body,
        grid=(E // _EBLK,),
        in_specs=[
            pl.BlockSpec((_EBLK, 1), lambda i: (i, 0)),
            pl.BlockSpec((_EBLK, 2), lambda i: (i, 0)),
        ],
        out_specs=[
            pl.BlockSpec((_EBLK, 1), lambda i: (i, 0)),
            pl.BlockSpec((_EBLK, 1), lambda i: (i, 0)),
        ],
        out_shape=[
            jax.ShapeDtypeStruct((E, 1), jnp.float32),
            jax.ShapeDtypeStruct((E, 1), jnp.float32),
        ],
    )(d, e_type)
    return wlin.reshape(E), ndlin.reshape(E)


_RBLK = 1024

def _dense_body(part_ref, spart_ref, label_ref, wl1_ref, bl1_ref, wl2_ref,
                bl2_ref, wt3_ref, bt3_ref, wt4_ref, out_ref):
    f32 = jnp.float32
    n1h = part_ref[0] + part_ref[1]
    s2 = jnp.sum(spart_ref[...], axis=0)[:, None]
    w4r = jnp.maximum(wt4_ref[...], 0.0)
    v = lax.dot_general(wt3_ref[...], w4r,
                        (((1,), (0,)), ((), ())),
                        preferred_element_type=f32)
    t3 = lax.dot_general(s2, v, (((1,), (1,)), ((), ())),
                         preferred_element_type=f32)
    l1 = lax.dot_general(label_ref[...], wl1_ref[...],
                         (((1,), (1,)), ((), ())),
                         preferred_element_type=f32)
    l2 = lax.dot_general(n1h, wl2_ref[...],
                         (((1,), (1,)), ((), ())),
                         preferred_element_type=f32)
    bias = (bl1_ref[...] + bl2_ref[...] + bt3_ref[...])[None, :]
    out_ref[...] = jnp.maximum(l1 + l2 + t3 + bias, 0.0)


def _dense(part, spart, label, W_l1, b_l1, W_l2, b_l2, W_t3, b_t3, W_t4):
    full2 = lambda i: (0, 0)
    return pl.pallas_call(
        _dense_body,
        grid=(pl.cdiv(N, _RBLK),),
        in_specs=[
            pl.BlockSpec((NC, _RBLK, H), lambda i: (0, i, 0)),
            pl.BlockSpec((NW, _RBLK), lambda i: (0, i)),
            pl.BlockSpec((_RBLK, K), lambda i: (i, 0)),
            pl.BlockSpec((H, K), full2),
            pl.BlockSpec((H,), lambda i: (0,)),
            pl.BlockSpec((H, H), full2),
            pl.BlockSpec((H,), lambda i: (0,)),
            pl.BlockSpec((H, H), full2),
            pl.BlockSpec((H,), lambda i: (0,)),
            pl.BlockSpec((H, 1), full2),
        ],
        out_specs=pl.BlockSpec((_RBLK, H), lambda i: (i, 0)),
        out_shape=jax.ShapeDtypeStruct((N, H), jnp.float32),
    )(part, spart, label, W_l1, b_l1, W_l2, b_l2, W_t3, b_t3, W_t4)


def kernel(h, label, d, e_type, src, dst, W_l1, b_l1, W_l2, b_l2,
           W_t3, b_t3, W_t4, b_t4):
    del b_t4
    wlin, ndlin = _preprocess(d, e_type)
    part, spart = _sc_call(h, src, dst, wlin, ndlin)
    return _dense(part, spart, label, W_l1, b_l1, W_l2, b_l2, W_t3, b_t3, W_t4)


# speedup vs baseline: 3.8092x; 1.4785x over previous
import functools

import jax
import jax.numpy as jnp
from jax import lax
from jax.experimental import pallas as pl
from jax.experimental.pallas import tpu as pltpu
from jax.experimental.pallas import tpu_sc as plsc

N = 10000
E = 320000
K = 10
H = 128

NC = 2
NS = 16
NW = NC * NS
L = 16

EPW = E // NW
B = 80
NBATCH = EPW // B
N_PAD = 10240
SROWS = N_PAD // NS


PAIRS = (NBATCH - 1) // 2


def _sc_body(h_hbm, src_hbm, dst_hbm, wlin_hbm, ndlin_hbm, part_hbm, spart_hbm,
             srcv0, dstv0, wv0, nd0, rows0,
             srcv1, dstv1, wv1, nd1, rows1,
             s2loc, acc, semf0, semf1, semg0, semg1):
    c = lax.axis_index("c")
    s = lax.axis_index("s")
    wid = c * NS + s

    sets = [
        (srcv0, dstv0, wv0, nd0, rows0, semf0, semg0),
        (srcv1, dstv1, wv1, nd1, rows1, semf1, semg1),
    ]

    def _zero_body(i, _):
        for j in range(H // L):
            rows0[i, pl.ds(j * L, L)] = jnp.zeros((L,), jnp.float32)
        return 0
    lax.fori_loop(0, B, _zero_body, 0)

    def _zero_s2(i, _):
        s2loc[pl.ds(i * L, L)] = jnp.zeros((L,), jnp.float32)
        return 0
    lax.fori_loop(0, N_PAD // L, _zero_s2, 0)

    for t in range(SROWS // B):
        pltpu.sync_copy(rows0, acc.at[pl.ds(s * SROWS + t * B, B)])
    plsc.subcore_barrier()

    def fetch_start(S, it):
        base = wid * EPW + it * B
        pltpu.async_copy(src_hbm.at[pl.ds(base, B)], S[0], S[5])
        pltpu.async_copy(dst_hbm.at[pl.ds(base, B)], S[1], S[5])
        pltpu.async_copy(wlin_hbm.at[pl.ds(base, B)], S[2], S[5])
        pltpu.async_copy(ndlin_hbm.at[pl.ds(base, B)], S[3], S[5])

    def fetch_wait(S):
        pltpu.make_async_copy(src_hbm.at[pl.ds(0, B)], S[0], S[5]).wait()
        pltpu.make_async_copy(dst_hbm.at[pl.ds(0, B)], S[1], S[5]).wait()
        pltpu.make_async_copy(wlin_hbm.at[pl.ds(0, B)], S[2], S[5]).wait()
        pltpu.make_async_copy(ndlin_hbm.at[pl.ds(0, B)], S[3], S[5]).wait()

    def gather_start(S):
        pltpu.async_copy(h_hbm.at[S[0]], S[4], S[6])

    def gather_wait(S):
        pltpu.make_async_copy(h_hbm.at[S[0]], S[4], S[6]).wait()

    def compute(S):
        dstv_, wv_, nd_, rows_ = S[1], S[2], S[3], S[4]

        def _s2_body(k, _):
            idx16 = dstv_[pl.ds(k * L, L)]
            v16 = nd_[pl.ds(k * L, L)]
            plsc.addupdate_scatter(s2loc, [idx16], v16)
            return 0
        lax.fori_loop(0, B // L, _s2_body, 0)

        gather_wait(S)

        @plsc.parallel_loop(0, B, unroll=4)
        def _scale(i):
            ridx = jnp.full((L,), i, dtype=jnp.int32)
            wv = plsc.load_gather(wv_, [ridx])
            for j in range(H // L):
                rows_[i, pl.ds(j * L, L)] = rows_[i, pl.ds(j * L, L)] * wv

        pltpu.sync_copy(rows_, acc.at[dstv_], add=True)

    fetch_start(sets[0], 0)
    fetch_wait(sets[0])
    gather_start(sets[0])
    fetch_start(sets[1], 1)

    def _pair_body(k, _):
        g = 2 * k
        S0, S1 = sets
        fetch_wait(S1)
        gather_start(S1)
        compute(S0)
        fetch_start(S0, g + 2)
        fetch_wait(S0)
        gather_start(S0)
        compute(S1)
        fetch_start(S1, jnp.minimum(g + 3, NBATCH - 1))
        return 0
    lax.fori_loop(0, PAIRS, _pair_body, 0)

    compute(sets[0])
    fetch_wait(sets[1])

    plsc.subcore_barrier()

    r0 = s * SROWS
    pltpu.sync_copy(acc.at[pl.ds(r0, SROWS)], part_hbm.at[c, pl.ds(r0, SROWS)])
    pltpu.sync_copy(s2loc, spart_hbm.at[wid])


_sc_call = pl.kernel(
    _sc_body,
    out_type=[
        jax.ShapeDtypeStruct((NC, N_PAD, H), jnp.float32),
        jax.ShapeDtypeStruct((NW, N_PAD), jnp.float32),
    ],
    mesh=plsc.VectorSubcoreMesh(core_axis_name="c", subcore_axis_name="s"),
    compiler_params=pltpu.CompilerParams(needs_layout_passes=False),
    scratch_types=(
        [pltpu.VMEM((B,), jnp.int32),
         pltpu.VMEM((B,), jnp.int32),
         pltpu.VMEM((B,), jnp.float32),
         pltpu.VMEM((B,), jnp.float32),
         pltpu.VMEM((B, H), jnp.float32)] * 2
        + [
            pltpu.VMEM((N_PAD,), jnp.float32),
            pltpu.VMEM_SHARED((N_PAD, H), jnp.float32),
            pltpu.SemaphoreType.DMA,
            pltpu.SemaphoreType.DMA,
            pltpu.SemaphoreType.DMA,
            pltpu.SemaphoreType.DMA,
        ]
    ),
)


_EBLK = 2000

def _pre_body(d_ref, et_ref, wlin_ref, ndlin_ref):
    et0 = et_ref[:, 0:1]
    nd = d_ref[...] * et0
    ndlin_ref[...] = nd
    wlin_ref[...] = nd * et0


def _preprocess(d, e_type):
    wlin, ndlin = pl.pallas_call(
        _pre_body,
        grid=(E // _EBLK,),
        in_specs=[
            pl.BlockSpec((_EBLK, 1), lambda i: (i, 0)),
            pl.BlockSpec((_EBLK, 2), lambda i: (i, 0)),
        ],
        out_specs=[
            pl.BlockSpec((_EBLK, 1), lambda i: (i, 0)),
            pl.BlockSpec((_EBLK, 1), lambda i: (i, 0)),
        ],
        out_shape=[
            jax.ShapeDtypeStruct((E, 1), jnp.float32),
            jax.ShapeDtypeStruct((E, 1), jnp.float32),
        ],
    )(d, e_type)
    return wlin.reshape(E), ndlin.reshape(E)


_RBLK = 1024

def _dense_body(part_ref, spart_ref, label_ref, wl1_ref, bl1_ref, wl2_ref,
                bl2_ref, wt3_ref, bt3_ref, wt4_ref, out_ref):
    f32 = jnp.float32
    n1h = part_ref[0] + part_ref[1]
    s2 = jnp.sum(spart_ref[...], axis=0)[:, None]
    w4r = jnp.maximum(wt4_ref[...], 0.0)
    v = lax.dot_general(wt3_ref[...], w4r,
                        (((1,), (0,)), ((), ())),
                        preferred_element_type=f32)
    t3 = lax.dot_general(s2, v, (((1,), (1,)), ((), ())),
                         preferred_element_type=f32)
    l1 = lax.dot_general(label_ref[...], wl1_ref[...],
                         (((1,), (1,)), ((), ())),
                         preferred_element_type=f32)
    l2 = lax.dot_general(n1h, wl2_ref[...],
                         (((1,), (1,)), ((), ())),
                         preferred_element_type=f32)
    bias = (bl1_ref[...] + bl2_ref[...] + bt3_ref[...])[None, :]
    out_ref[...] = jnp.maximum(l1 + l2 + t3 + bias, 0.0)


def _dense(part, spart, label, W_l1, b_l1, W_l2, b_l2, W_t3, b_t3, W_t4):
    full2 = lambda i: (0, 0)
    return pl.pallas_call(
        _dense_body,
        grid=(pl.cdiv(N, _RBLK),),
        in_specs=[
            pl.BlockSpec((NC, _RBLK, H), lambda i: (0, i, 0)),
            pl.BlockSpec((NW, _RBLK), lambda i: (0, i)),
            pl.BlockSpec((_RBLK, K), lambda i: (i, 0)),
            pl.BlockSpec((H, K), full2),
            pl.BlockSpec((H,), lambda i: (0,)),
            pl.BlockSpec((H, H), full2),
            pl.BlockSpec((H,), lambda i: (0,)),
            pl.BlockSpec((H, H), full2),
            pl.BlockSpec((H,), lambda i: (0,)),
            pl.BlockSpec((H, 1), full2),
        ],
        out_specs=pl.BlockSpec((_RBLK, H), lambda i: (i, 0)),
        out_shape=jax.ShapeDtypeStruct((N, H), jnp.float32),
    )(part, spart, label, W_l1, b_l1, W_l2, b_l2, W_t3, b_t3, W_t4)


def kernel(h, label, d, e_type, src, dst, W_l1, b_l1, W_l2, b_l2,
           W_t3, b_t3, W_t4, b_t4):
    del b_t4
    wlin, ndlin = _preprocess(d, e_type)
    part, spart = _sc_call(h, src, dst, wlin, ndlin)
    return _dense(part, spart, label, W_l1, b_l1, W_l2, b_l2, W_t3, b_t3, W_t4)


# speedup vs baseline: 10.3383x; 2.7140x over previous
import functools

import jax
import jax.numpy as jnp
from jax import lax
from jax.experimental import pallas as pl
from jax.experimental.pallas import tpu as pltpu
from jax.experimental.pallas import tpu_sc as plsc

N = 10000
E = 320000
K = 10
H = 128

NC = 2
NS = 16
NW = NC * NS
L = 16

EPW = E // NW
B = 80
NBATCH = EPW // B
N_PAD = 10240
SROWS = N_PAD // NS


PAIRS = (NBATCH - 1) // 2


def _sc_body(h_hbm, src_hbm, dst_hbm, d_hbm, et0_hbm, part_hbm, spart_hbm,
             srcv0, dstv0, wv0, nd0, rows0,
             srcv1, dstv1, wv1, nd1, rows1,
             s2loc, acc, semf0, semf1, semg0, semg1):
    c = lax.axis_index("c")
    s = lax.axis_index("s")
    wid = c * NS + s

    sets = [
        (srcv0, dstv0, wv0, nd0, rows0, semf0, semg0),
        (srcv1, dstv1, wv1, nd1, rows1, semf1, semg1),
    ]

    def _zero_body(i, _):
        for j in range(H // L):
            rows0[i, pl.ds(j * L, L)] = jnp.zeros((L,), jnp.float32)
        return 0
    lax.fori_loop(0, B, _zero_body, 0)

    def _zero_s2(i, _):
        s2loc[pl.ds(i * L, L)] = jnp.zeros((L,), jnp.float32)
        return 0
    lax.fori_loop(0, N_PAD // L, _zero_s2, 0)

    for t in range(SROWS // B):
        pltpu.sync_copy(rows0, acc.at[pl.ds(s * SROWS + t * B, B)])
    plsc.subcore_barrier()

    def fetch_start(S, it):
        base = wid * EPW + it * B
        pltpu.async_copy(src_hbm.at[pl.ds(base, B)], S[0], S[5])
        pltpu.async_copy(dst_hbm.at[pl.ds(base, B)], S[1], S[5])
        pltpu.async_copy(d_hbm.at[pl.ds(base, B)], S[2], S[5])
        pltpu.async_copy(et0_hbm.at[pl.ds(base, B)], S[3], S[5])

    def fetch_wait(S):
        pltpu.make_async_copy(src_hbm.at[pl.ds(0, B)], S[0], S[5]).wait()
        pltpu.make_async_copy(dst_hbm.at[pl.ds(0, B)], S[1], S[5]).wait()
        pltpu.make_async_copy(d_hbm.at[pl.ds(0, B)], S[2], S[5]).wait()
        pltpu.make_async_copy(et0_hbm.at[pl.ds(0, B)], S[3], S[5]).wait()

    def gather_start(S):
        pltpu.async_copy(h_hbm.at[S[0]], S[4], S[6])

    def gather_wait(S):
        pltpu.make_async_copy(h_hbm.at[S[0]], S[4], S[6]).wait()

    def compute(S):
        dstv_, wv_, nd_, rows_ = S[1], S[2], S[3], S[4]

        def _s2_body(k, _):
            dd = wv_[pl.ds(k * L, L)]
            ee = nd_[pl.ds(k * L, L)]
            nd16 = dd * ee
            wv_[pl.ds(k * L, L)] = nd16 * ee
            idx16 = dstv_[pl.ds(k * L, L)]
            plsc.addupdate_scatter(s2loc, [idx16], nd16)
            return 0
        lax.fori_loop(0, B // L, _s2_body, 0)

        gather_wait(S)

        @plsc.parallel_loop(0, B, unroll=4)
        def _scale(i):
            ridx = jnp.full((L,), i, dtype=jnp.int32)
            wv = plsc.load_gather(wv_, [ridx])
            for j in range(H // L):
                rows_[i, pl.ds(j * L, L)] = rows_[i, pl.ds(j * L, L)] * wv

        pltpu.sync_copy(rows_, acc.at[dstv_], add=True)

    fetch_start(sets[0], 0)
    fetch_wait(sets[0])
    gather_start(sets[0])
    fetch_start(sets[1], 1)

    def _pair_body(k, _):
        g = 2 * k
        S0, S1 = sets
        fetch_wait(S1)
        gather_start(S1)
        compute(S0)
        fetch_start(S0, g + 2)
        fetch_wait(S0)
        gather_start(S0)
        compute(S1)
        fetch_start(S1, jnp.minimum(g + 3, NBATCH - 1))
        return 0
    lax.fori_loop(0, PAIRS, _pair_body, 0)

    compute(sets[0])
    fetch_wait(sets[1])

    plsc.subcore_barrier()

    r0 = s * SROWS
    pltpu.sync_copy(acc.at[pl.ds(r0, SROWS)], part_hbm.at[c, pl.ds(r0, SROWS)])
    pltpu.sync_copy(s2loc, spart_hbm.at[wid])


_sc_call = pl.kernel(
    _sc_body,
    out_type=[
        jax.ShapeDtypeStruct((NC, N_PAD, H), jnp.float32),
        jax.ShapeDtypeStruct((NW, N_PAD), jnp.float32),
    ],
    mesh=plsc.VectorSubcoreMesh(core_axis_name="c", subcore_axis_name="s"),
    compiler_params=pltpu.CompilerParams(needs_layout_passes=False),
    scratch_types=(
        [pltpu.VMEM((B,), jnp.int32),
         pltpu.VMEM((B,), jnp.int32),
         pltpu.VMEM((B,), jnp.float32),
         pltpu.VMEM((B,), jnp.float32),
         pltpu.VMEM((B, H), jnp.float32)] * 2
        + [
            pltpu.VMEM((N_PAD,), jnp.float32),
            pltpu.VMEM_SHARED((N_PAD, H), jnp.float32),
            pltpu.SemaphoreType.DMA,
            pltpu.SemaphoreType.DMA,
            pltpu.SemaphoreType.DMA,
            pltpu.SemaphoreType.DMA,
        ]
    ),
)


_RBLK = 1024

def _dense_body(part_ref, spart_ref, label_ref, wl1_ref, bl1_ref, wl2_ref,
                bl2_ref, wt3_ref, bt3_ref, wt4_ref, out_ref):
    f32 = jnp.float32
    n1h = part_ref[0] + part_ref[1]
    s2 = jnp.sum(spart_ref[...], axis=0)[:, None]
    w4r = jnp.maximum(wt4_ref[...], 0.0)
    v = lax.dot_general(wt3_ref[...], w4r,
                        (((1,), (0,)), ((), ())),
                        preferred_element_type=f32)
    t3 = lax.dot_general(s2, v, (((1,), (1,)), ((), ())),
                         preferred_element_type=f32)
    l1 = lax.dot_general(label_ref[...], wl1_ref[...],
                         (((1,), (1,)), ((), ())),
                         preferred_element_type=f32)
    l2 = lax.dot_general(n1h, wl2_ref[...],
                         (((1,), (1,)), ((), ())),
                         preferred_element_type=f32)
    bias = (bl1_ref[...] + bl2_ref[...] + bt3_ref[...])[None, :]
    out_ref[...] = jnp.maximum(l1 + l2 + t3 + bias, 0.0)


def _dense(part, spart, label, W_l1, b_l1, W_l2, b_l2, W_t3, b_t3, W_t4):
    full2 = lambda i: (0, 0)
    return pl.pallas_call(
        _dense_body,
        grid=(pl.cdiv(N, _RBLK),),
        in_specs=[
            pl.BlockSpec((NC, _RBLK, H), lambda i: (0, i, 0)),
            pl.BlockSpec((NW, _RBLK), lambda i: (0, i)),
            pl.BlockSpec((_RBLK, K), lambda i: (i, 0)),
            pl.BlockSpec((H, K), full2),
            pl.BlockSpec((H,), lambda i: (0,)),
            pl.BlockSpec((H, H), full2),
            pl.BlockSpec((H,), lambda i: (0,)),
            pl.BlockSpec((H, H), full2),
            pl.BlockSpec((H,), lambda i: (0,)),
            pl.BlockSpec((H, 1), full2),
        ],
        out_specs=pl.BlockSpec((_RBLK, H), lambda i: (i, 0)),
        out_shape=jax.ShapeDtypeStruct((N, H), jnp.float32),
    )(part, spart, label, W_l1, b_l1, W_l2, b_l2, W_t3, b_t3, W_t4)


def kernel(h, label, d, e_type, src, dst, W_l1, b_l1, W_l2, b_l2,
           W_t3, b_t3, W_t4, b_t4):
    del b_t4
    part, spart = _sc_call(h, src, dst, d.reshape(E), e_type[:, 0])
    return _dense(part, spart, label, W_l1, b_l1, W_l2, b_l2, W_t3, b_t3, W_t4)


# speedup vs baseline: 12.5687x; 1.2158x over previous
import functools

import jax
import jax.numpy as jnp
from jax import lax
from jax.experimental import pallas as pl
from jax.experimental.pallas import tpu as pltpu
from jax.experimental.pallas import tpu_sc as plsc

N = 10000
E = 320000
K = 10
H = 128

NC = 2
NS = 16
NW = NC * NS
L = 16

EPW = E // NW
B = 80
NBATCH = EPW // B
N_PAD = 10240
SROWS = N_PAD // NS


TRIPS = NBATCH // 3


def _sc_body(h_hbm, src_hbm, dst_hbm, d_hbm, et0_hbm, part_hbm, spart_hbm,
             srcv0, dstv0, wv0, nd0, rows0,
             srcv1, dstv1, wv1, nd1, rows1,
             srcv2, dstv2, wv2, nd2, rows2,
             s2loc, acc,
             semf0, semf1, semf2, semg0, semg1, semg2, sems0, sems1, sems2):
    c = lax.axis_index("c")
    s = lax.axis_index("s")
    wid = c * NS + s

    sets = [
        (srcv0, dstv0, wv0, nd0, rows0, semf0, semg0, sems0),
        (srcv1, dstv1, wv1, nd1, rows1, semf1, semg1, sems1),
        (srcv2, dstv2, wv2, nd2, rows2, semf2, semg2, sems2),
    ]

    def _zero_body(i, _):
        for j in range(H // L):
            rows0[i, pl.ds(j * L, L)] = jnp.zeros((L,), jnp.float32)
            rows2[i, pl.ds(j * L, L)] = jnp.zeros((L,), jnp.float32)
        return 0
    lax.fori_loop(0, B, _zero_body, 0)

    def _zero_s2(i, _):
        s2loc[pl.ds(i * L, L)] = jnp.zeros((L,), jnp.float32)
        return 0
    lax.fori_loop(0, N_PAD // L, _zero_s2, 0)

    for k in range(B // L):
        dstv2[pl.ds(k * L, L)] = jnp.zeros((L,), jnp.int32)

    for t in range(SROWS // B):
        pltpu.sync_copy(rows0, acc.at[pl.ds(s * SROWS + t * B, B)])
    plsc.subcore_barrier()

    def fetch_start(S, it):
        base = wid * EPW + it * B
        pltpu.async_copy(src_hbm.at[pl.ds(base, B)], S[0], S[5])
        pltpu.async_copy(dst_hbm.at[pl.ds(base, B)], S[1], S[5])
        pltpu.async_copy(d_hbm.at[pl.ds(base, B)], S[2], S[5])
        pltpu.async_copy(et0_hbm.at[pl.ds(base, B)], S[3], S[5])

    def fetch_wait(S):
        pltpu.make_async_copy(src_hbm.at[pl.ds(0, B)], S[0], S[5]).wait()
        pltpu.make_async_copy(dst_hbm.at[pl.ds(0, B)], S[1], S[5]).wait()
        pltpu.make_async_copy(d_hbm.at[pl.ds(0, B)], S[2], S[5]).wait()
        pltpu.make_async_copy(et0_hbm.at[pl.ds(0, B)], S[3], S[5]).wait()

    def gather_start(S):
        pltpu.async_copy(h_hbm.at[S[0]], S[4], S[6])

    def gather_wait(S):
        pltpu.make_async_copy(h_hbm.at[S[0]], S[4], S[6]).wait()

    def scatter_start(S):
        pltpu.async_copy(S[4], acc.at[S[1]], S[7], add=True)

    def scatter_wait(S):
        pltpu.make_async_copy(S[4], acc.at[S[1]], S[7]).wait()

    def compute_ns(S):
        dstv_, wv_, nd_, rows_ = S[1], S[2], S[3], S[4]

        def _s2_body(k, _):
            dd = wv_[pl.ds(k * L, L)]
            ee = nd_[pl.ds(k * L, L)]
            nd16 = dd * ee
            wv_[pl.ds(k * L, L)] = nd16 * ee
            idx16 = dstv_[pl.ds(k * L, L)]
            plsc.addupdate_scatter(s2loc, [idx16], nd16)
            return 0
        lax.fori_loop(0, B // L, _s2_body, 0)

        gather_wait(S)

        @plsc.parallel_loop(0, B, unroll=4)
        def _scale(i):
            ridx = jnp.full((L,), i, dtype=jnp.int32)
            wv = plsc.load_gather(wv_, [ridx])
            for j in range(H // L):
                rows_[i, pl.ds(j * L, L)] = rows_[i, pl.ds(j * L, L)] * wv

    fetch_start(sets[0], 0)
    fetch_wait(sets[0])
    gather_start(sets[0])
    fetch_start(sets[1], 1)
    scatter_start(sets[2])

    def _trip_body(k, _):
        for j in range(3):
            b = 3 * k + j
            S = sets[j]
            Snext = sets[(j + 1) % 3]
            Sprev = sets[(j + 2) % 3]
            fetch_wait(Snext)
            gather_start(Snext)
            compute_ns(S)
            scatter_start(S)
            scatter_wait(Sprev)
            fetch_start(Sprev, b + 2)
        return 0
    lax.fori_loop(0, TRIPS, _trip_body, 0)

    fetch_wait(sets[1])
    gather_start(sets[1])
    compute_ns(sets[0])
    scatter_start(sets[0])
    compute_ns(sets[1])
    scatter_start(sets[1])
    scatter_wait(sets[2])
    scatter_wait(sets[0])
    scatter_wait(sets[1])

    plsc.subcore_barrier()

    r0 = s * SROWS
    pltpu.sync_copy(acc.at[pl.ds(r0, SROWS)], part_hbm.at[c, pl.ds(r0, SROWS)])
    pltpu.sync_copy(s2loc, spart_hbm.at[wid])


_sc_call = pl.kernel(
    _sc_body,
    out_type=[
        jax.ShapeDtypeStruct((NC, N_PAD, H), jnp.float32),
        jax.ShapeDtypeStruct((NW, N_PAD), jnp.float32),
    ],
    mesh=plsc.VectorSubcoreMesh(core_axis_name="c", subcore_axis_name="s"),
    compiler_params=pltpu.CompilerParams(needs_layout_passes=False),
    scratch_types=(
        [pltpu.VMEM((B,), jnp.int32),
         pltpu.VMEM((B,), jnp.int32),
         pltpu.VMEM((B,), jnp.float32),
         pltpu.VMEM((B,), jnp.float32),
         pltpu.VMEM((B, H), jnp.float32)] * 3
        + [
            pltpu.VMEM((N_PAD,), jnp.float32),
            pltpu.VMEM_SHARED((N_PAD, H), jnp.float32),
        ]
        + [pltpu.SemaphoreType.DMA] * 9
    ),
)


_RBLK = 1024

def _dense_body(part_ref, spart_ref, label_ref, wl1_ref, bl1_ref, wl2_ref,
                bl2_ref, wt3_ref, bt3_ref, wt4_ref, out_ref):
    f32 = jnp.float32
    n1h = part_ref[0] + part_ref[1]
    s2 = jnp.sum(spart_ref[...], axis=0)[:, None]
    w4r = jnp.maximum(wt4_ref[...], 0.0)
    v = lax.dot_general(wt3_ref[...], w4r,
                        (((1,), (0,)), ((), ())),
                        preferred_element_type=f32)
    t3 = lax.dot_general(s2, v, (((1,), (1,)), ((), ())),
                         preferred_element_type=f32)
    l1 = lax.dot_general(label_ref[...], wl1_ref[...],
                         (((1,), (1,)), ((), ())),
                         preferred_element_type=f32)
    l2 = lax.dot_general(n1h, wl2_ref[...],
                         (((1,), (1,)), ((), ())),
                         preferred_element_type=f32)
    bias = (bl1_ref[...] + bl2_ref[...] + bt3_ref[...])[None, :]
    out_ref[...] = jnp.maximum(l1 + l2 + t3 + bias, 0.0)


def _dense(part, spart, label, W_l1, b_l1, W_l2, b_l2, W_t3, b_t3, W_t4):
    full2 = lambda i: (0, 0)
    return pl.pallas_call(
        _dense_body,
        grid=(pl.cdiv(N, _RBLK),),
        in_specs=[
            pl.BlockSpec((NC, _RBLK, H), lambda i: (0, i, 0)),
            pl.BlockSpec((NW, _RBLK), lambda i: (0, i)),
            pl.BlockSpec((_RBLK, K), lambda i: (i, 0)),
            pl.BlockSpec((H, K), full2),
            pl.BlockSpec((H,), lambda i: (0,)),
            pl.BlockSpec((H, H), full2),
            pl.BlockSpec((H,), lambda i: (0,)),
            pl.BlockSpec((H, H), full2),
            pl.BlockSpec((H,), lambda i: (0,)),
            pl.BlockSpec((H, 1), full2),
        ],
        out_specs=pl.BlockSpec((_RBLK, H), lambda i: (i, 0)),
        out_shape=jax.ShapeDtypeStruct((N, H), jnp.float32),
    )(part, spart, label, W_l1, b_l1, W_l2, b_l2, W_t3, b_t3, W_t4)


def kernel(h, label, d, e_type, src, dst, W_l1, b_l1, W_l2, b_l2,
           W_t3, b_t3, W_t4, b_t4):
    del b_t4
    part, spart = _sc_call(h, src, dst, d.reshape(E), e_type[:, 0])
    return _dense(part, spart, label, W_l1, b_l1, W_l2, b_l2, W_t3, b_t3, W_t4)


# speedup vs baseline: 12.5723x; 1.0003x over previous
import jax
import jax.numpy as jnp
from jax import lax
from jax.experimental import pallas as pl
from jax.experimental.pallas import tpu as pltpu
from jax.experimental.pallas import tpu_sc as plsc

N = 10000
E = 320000
K = 10
H = 128

NC = 2
NS = 16
NW = NC * NS
L = 16

EPW = E // NW
B = 80
NBATCH = EPW // B
N_PAD = 10240
SROWS = N_PAD // NS


TRIPS = NBATCH // 3


def _sc_body(h_hbm, src_hbm, dst_hbm, d_hbm, et0_hbm, part_hbm, spart_hbm,
             srcv0, dstv0, wv0, nd0, rows0,
             srcv1, dstv1, wv1, nd1, rows1,
             srcv2, dstv2, wv2, nd2, rows2,
             s2loc, acc,
             semf0, semf1, semf2, semg0, semg1, semg2, sems0, sems1, sems2):
    c = lax.axis_index("c")
    s = lax.axis_index("s")
    wid = c * NS + s

    sets = [
        (srcv0, dstv0, wv0, nd0, rows0, semf0, semg0, sems0),
        (srcv1, dstv1, wv1, nd1, rows1, semf1, semg1, sems1),
        (srcv2, dstv2, wv2, nd2, rows2, semf2, semg2, sems2),
    ]

    def _zero_body(i, _):
        for j in range(H // L):
            rows0[i, pl.ds(j * L, L)] = jnp.zeros((L,), jnp.float32)
            rows2[i, pl.ds(j * L, L)] = jnp.zeros((L,), jnp.float32)
        return 0
    lax.fori_loop(0, B, _zero_body, 0)

    def _zero_s2(i, _):
        s2loc[pl.ds(i * L, L)] = jnp.zeros((L,), jnp.float32)
        return 0
    lax.fori_loop(0, N_PAD // L, _zero_s2, 0)

    for k in range(B // L):
        dstv2[pl.ds(k * L, L)] = jnp.zeros((L,), jnp.int32)

    for t in range(SROWS // B):
        pltpu.sync_copy(rows0, acc.at[pl.ds(s * SROWS + t * B, B)])
    plsc.subcore_barrier()

    def fetch_start(S, it):
        base = wid * EPW + it * B
        pltpu.async_copy(src_hbm.at[pl.ds(base, B)], S[0], S[5])
        pltpu.async_copy(dst_hbm.at[pl.ds(base, B)], S[1], S[5])
        pltpu.async_copy(d_hbm.at[pl.ds(base, B)], S[2], S[5])
        pltpu.async_copy(et0_hbm.at[pl.ds(base, B)], S[3], S[5])

    def fetch_wait(S):
        pltpu.make_async_copy(src_hbm.at[pl.ds(0, B)], S[0], S[5]).wait()
        pltpu.make_async_copy(dst_hbm.at[pl.ds(0, B)], S[1], S[5]).wait()
        pltpu.make_async_copy(d_hbm.at[pl.ds(0, B)], S[2], S[5]).wait()
        pltpu.make_async_copy(et0_hbm.at[pl.ds(0, B)], S[3], S[5]).wait()

    def gather_start(S):
        pltpu.async_copy(h_hbm.at[S[0]], S[4], S[6])

    def gather_wait(S):
        pltpu.make_async_copy(h_hbm.at[S[0]], S[4], S[6]).wait()

    def scatter_start(S):
        pltpu.async_copy(S[4], acc.at[S[1]], S[7], add=True)

    def scatter_wait(S):
        pltpu.make_async_copy(S[4], acc.at[S[1]], S[7]).wait()

    def compute_ns(S):
        dstv_, wv_, nd_, rows_ = S[1], S[2], S[3], S[4]

        def _s2_body(k, _):
            dd = wv_[pl.ds(k * L, L)]
            ee = nd_[pl.ds(k * L, L)]
            nd16 = dd * ee
            wv_[pl.ds(k * L, L)] = nd16 * ee
            idx16 = dstv_[pl.ds(k * L, L)]
            plsc.addupdate_scatter(s2loc, [idx16], nd16)
            return 0
        lax.fori_loop(0, B // L, _s2_body, 0)

        gather_wait(S)

        @plsc.parallel_loop(0, B, unroll=4)
        def _scale(i):
            ridx = jnp.full((L,), i, dtype=jnp.int32)
            wv = plsc.load_gather(wv_, [ridx])
            for j in range(H // L):
                rows_[i, pl.ds(j * L, L)] = rows_[i, pl.ds(j * L, L)] * wv

    fetch_start(sets[0], 0)
    fetch_wait(sets[0])
    gather_start(sets[0])
    fetch_start(sets[1], 1)
    scatter_start(sets[2])

    def _trip_body(k, _):
        for j in range(3):
            b = 3 * k + j
            S = sets[j]
            Snext = sets[(j + 1) % 3]
            Sprev = sets[(j + 2) % 3]
            fetch_wait(Snext)
            gather_start(Snext)
            compute_ns(S)
            scatter_start(S)
            scatter_wait(Sprev)
            fetch_start(Sprev, b + 2)
        return 0
    lax.fori_loop(0, TRIPS, _trip_body, 0)

    fetch_wait(sets[1])
    gather_start(sets[1])
    compute_ns(sets[0])
    scatter_start(sets[0])
    compute_ns(sets[1])
    scatter_start(sets[1])
    scatter_wait(sets[2])
    scatter_wait(sets[0])
    scatter_wait(sets[1])

    plsc.subcore_barrier()

    r0 = s * SROWS
    pltpu.sync_copy(acc.at[pl.ds(r0, SROWS)], part_hbm.at[c, pl.ds(r0, SROWS)])
    pltpu.sync_copy(s2loc, spart_hbm.at[wid])


_sc_call = pl.kernel(
    _sc_body,
    out_type=[
        jax.ShapeDtypeStruct((NC, N_PAD, H), jnp.float32),
        jax.ShapeDtypeStruct((NW, N_PAD), jnp.float32),
    ],
    mesh=plsc.VectorSubcoreMesh(core_axis_name="c", subcore_axis_name="s"),
    compiler_params=pltpu.CompilerParams(needs_layout_passes=False),
    scratch_types=(
        [pltpu.VMEM((B,), jnp.int32),
         pltpu.VMEM((B,), jnp.int32),
         pltpu.VMEM((B,), jnp.float32),
         pltpu.VMEM((B,), jnp.float32),
         pltpu.VMEM((B, H), jnp.float32)] * 3
        + [
            pltpu.VMEM((N_PAD,), jnp.float32),
            pltpu.VMEM_SHARED((N_PAD, H), jnp.float32),
        ]
        + [pltpu.SemaphoreType.DMA] * 9
    ),
)


_RBLK = 1024

def _dense_body(part_ref, spart_ref, label_ref, wl1_ref, bl1_ref, wl2_ref,
                bl2_ref, wt3_ref, bt3_ref, wt4_ref, out_ref):
    f32 = jnp.float32
    n1h = part_ref[0] + part_ref[1]
    s2 = jnp.sum(spart_ref[...], axis=0)[:, None]
    w4r = jnp.maximum(wt4_ref[...], 0.0)
    v = lax.dot_general(wt3_ref[...], w4r,
                        (((1,), (0,)), ((), ())),
                        preferred_element_type=f32)
    t3 = lax.dot_general(s2, v, (((1,), (1,)), ((), ())),
                         preferred_element_type=f32)
    l1 = lax.dot_general(label_ref[...], wl1_ref[...],
                         (((1,), (1,)), ((), ())),
                         preferred_element_type=f32)
    l2 = lax.dot_general(n1h, wl2_ref[...],
                         (((1,), (1,)), ((), ())),
                         preferred_element_type=f32)
    bias = (bl1_ref[...] + bl2_ref[...] + bt3_ref[...])[None, :]
    out_ref[...] = jnp.maximum(l1 + l2 + t3 + bias, 0.0)


def _dense(part, spart, label, W_l1, b_l1, W_l2, b_l2, W_t3, b_t3, W_t4):
    full2 = lambda i: (0, 0)
    return pl.pallas_call(
        _dense_body,
        grid=(pl.cdiv(N, _RBLK),),
        in_specs=[
            pl.BlockSpec((NC, _RBLK, H), lambda i: (0, i, 0)),
            pl.BlockSpec((NW, _RBLK), lambda i: (0, i)),
            pl.BlockSpec((_RBLK, K), lambda i: (i, 0)),
            pl.BlockSpec((H, K), full2),
            pl.BlockSpec((H,), lambda i: (0,)),
            pl.BlockSpec((H, H), full2),
            pl.BlockSpec((H,), lambda i: (0,)),
            pl.BlockSpec((H, H), full2),
            pl.BlockSpec((H,), lambda i: (0,)),
            pl.BlockSpec((H, 1), full2),
        ],
        out_specs=pl.BlockSpec((_RBLK, H), lambda i: (i, 0)),
        out_shape=jax.ShapeDtypeStruct((N, H), jnp.float32),
    )(part, spart, label, W_l1, b_l1, W_l2, b_l2, W_t3, b_t3, W_t4)


def kernel(h, label, d, e_type, src, dst, W_l1, b_l1, W_l2, b_l2,
           W_t3, b_t3, W_t4, b_t4):
    del b_t4
    part, spart = _sc_call(h, src, dst, d.reshape(E), e_type[:, 0])
    return _dense(part, spart, label, W_l1, b_l1, W_l2, b_l2, W_t3, b_t3, W_t4)
